# Initial kernel scaffold; baseline (speedup 1.0000x reference)
import jax
import jax.numpy as jnp
from jax.experimental import pallas as pl


def kernel(z_e, embedding):
    raise NotImplementedError("write your pallas kernel here")


# speedup vs baseline: 1.2125x; 1.2125x over previous
import functools

import jax
import jax.numpy as jnp
from jax import lax
from jax.experimental import pallas as pl
from jax.experimental.pallas import tpu as pltpu
from jax.experimental.pallas import tpu_sc as plsc

KCODES = 8192
DIM = 32
BETA = 0.25
N_TILE = 512
K_CHUNK = 4096


def _vq_argmin_body(z_ref, emb_ref, idx_ref, lsum_ref):
    i = pl.program_id(0)
    nprog = pl.num_programs(0)
    z = z_ref[...]
    e = emb_ref[...]
    zsq = jnp.sum(z * z, axis=1, keepdims=True)
    esq = jnp.sum(e * e, axis=1)
    dot = lax.dot_general(z.astype(jnp.bfloat16), e.astype(jnp.bfloat16),
                          (((1,), (1,)), ((), ())),
                          preferred_element_type=jnp.float32)
    dist = (zsq + esq[None, :]) - 2.0 * dot

    carry_v = jnp.full((N_TILE,), jnp.inf, jnp.float32)
    carry_f = jnp.full((N_TILE,), jnp.inf, jnp.float32)
    carry_i = jnp.zeros((N_TILE,), jnp.int32)
    for c in range(KCODES // K_CHUNK):
        dc = dist[:, c * K_CHUNK:(c + 1) * K_CHUNK]
        m = jnp.min(dc, axis=1)
        iota = lax.broadcasted_iota(jnp.int32, dc.shape, 1)
        cand = jnp.where(dc <= m[:, None], iota, KCODES)
        idx_c = jnp.min(cand, axis=1) + c * K_CHUNK
        better = m < carry_v
        carry_i = jnp.where(better, idx_c, carry_i)
        carry_f = jnp.where(better, m, carry_f)
        carry_v = jnp.where(better, m, carry_v).astype(jnp.bfloat16).astype(jnp.float32)

    idx_ref[0, 0, :] = carry_i

    @pl.when(i == 0)
    def _():
        lsum_ref[...] = jnp.zeros((1, 1), jnp.float32)

    lsum_ref[...] += jnp.reshape(jnp.sum(carry_f), (1, 1))

    @pl.when(i == nprog - 1)
    def _():
        n_total = nprog * N_TILE
        lsum_ref[...] = lsum_ref[...] * ((1.0 + BETA) / (n_total * DIM))


def _vq_argmin(z_flat, embedding):
    n = z_flat.shape[0]
    nblocks = n // N_TILE
    return pl.pallas_call(
        _vq_argmin_body,
        grid=(nblocks,),
        in_specs=[
            pl.BlockSpec((N_TILE, DIM), lambda i: (i, 0)),
            pl.BlockSpec((KCODES, DIM), lambda i: (0, 0)),
        ],
        out_specs=[
            pl.BlockSpec((1, 1, N_TILE), lambda i: (i, 0, 0)),
            pl.BlockSpec((1, 1), lambda i: (0, 0)),
        ],
        out_shape=[
            jax.ShapeDtypeStruct((nblocks, 1, N_TILE), jnp.int32),
            jax.ShapeDtypeStruct((1, 1), jnp.float32),
        ],
    )(z_flat, embedding)


@functools.cache
def _make_sc_gather(n_total):
    info = plsc.get_sparse_core_info()
    nw = info.num_cores * info.num_subcores
    b_per_w = n_total // nw
    mesh = plsc.VectorSubcoreMesh(core_axis_name="c", subcore_axis_name="s")

    @functools.partial(
        pl.kernel, mesh=mesh,
        compiler_params=pltpu.CompilerParams(use_tc_tiling_on_sc=False),
        out_type=jax.ShapeDtypeStruct((n_total, DIM), jnp.float32),
        scratch_types=[
            pltpu.VMEM((b_per_w,), jnp.int32),
            pltpu.VMEM((b_per_w, DIM), jnp.float32),
            pltpu.SemaphoreType.DMA,
        ],
    )
    def gather_kernel(idx_hbm, table_hbm, out_hbm, idx_v, rows_v, sem):
        wid = lax.axis_index("s") * info.num_cores + lax.axis_index("c")
        base = wid * b_per_w
        pltpu.sync_copy(idx_hbm.at[pl.ds(base, b_per_w)], idx_v)
        pltpu.async_copy(table_hbm.at[idx_v], rows_v, sem).wait()
        pltpu.sync_copy(rows_v, out_hbm.at[pl.ds(base, b_per_w)])

    return gather_kernel


def kernel(z_e, embedding):
    b, d, h, w = z_e.shape
    n = b * h * w
    z_flat = jnp.transpose(z_e, (0, 2, 3, 1)).reshape(n, d)
    idx3, lsum = _vq_argmin(z_flat, embedding)
    indices = idx3.reshape(n)
    z_q_flat = _make_sc_gather(n)(indices, embedding)
    z_q = jnp.transpose(z_q_flat.reshape(b, h, w, d), (0, 3, 1, 2))
    return (z_q, lsum[0, 0], indices.reshape(b, h, w))


# speedup vs baseline: 1.5313x; 1.2629x over previous
import functools

import jax
import jax.numpy as jnp
from jax import lax
from jax.experimental import pallas as pl
from jax.experimental.pallas import tpu as pltpu
from jax.experimental.pallas import tpu_sc as plsc

KCODES = 8192
DIM = 32
BETA = 0.25
HW_TILE = 512
K_CHUNK = 4096


def _vq_argmin_body(z_ref, emb_ref, idx_ref, lsum_ref):
    bi = pl.program_id(0)
    ji = pl.program_id(1)
    nb = pl.num_programs(0)
    nj = pl.num_programs(1)
    z = z_ref[0]
    e = emb_ref[...]
    zsq = jnp.sum(z * z, axis=0)
    esq = jnp.sum(e * e, axis=1)
    zb2 = (z * 2.0).astype(jnp.bfloat16)
    eb = e.astype(jnp.bfloat16)
    conv = lax.dot_general(eb, zb2, (((1,), (0,)), ((), ())),
                           preferred_element_type=jnp.float32)

    carry_v = jnp.full((HW_TILE,), jnp.inf, jnp.float32)
    carry_f = jnp.full((HW_TILE,), jnp.inf, jnp.float32)
    carry_i = jnp.zeros((HW_TILE,), jnp.int32)
    for c in range(KCODES // K_CHUNK):
        sl = slice(c * K_CHUNK, (c + 1) * K_CHUNK)
        dc = esq[sl, None] - conv[sl, :]
        mp = jnp.min(dc, axis=0)
        iota = lax.broadcasted_iota(jnp.int32, dc.shape, 0)
        cand = jnp.where(dc <= mp[None, :], iota, KCODES)
        idx_c = jnp.min(cand, axis=0) + c * K_CHUNK
        m = zsq + mp
        better = m < carry_v
        carry_i = jnp.where(better, idx_c, carry_i)
        carry_f = jnp.where(better, m, carry_f)
        carry_v = jnp.where(better, m, carry_v).astype(jnp.bfloat16).astype(jnp.float32)

    idx_ref[0, 0, 0, :] = carry_i

    @pl.when(jnp.logical_and(bi == 0, ji == 0))
    def _():
        lsum_ref[...] = jnp.zeros((1, 1), jnp.float32)

    lsum_ref[...] += jnp.reshape(jnp.sum(carry_f), (1, 1))

    @pl.when(jnp.logical_and(bi == nb - 1, ji == nj - 1))
    def _():
        n_total = nb * nj * HW_TILE
        lsum_ref[...] = lsum_ref[...] * ((1.0 + BETA) / (n_total * DIM))


def _vq_argmin(z3, embedding):
    b, d, hw = z3.shape
    nj = hw // HW_TILE
    return pl.pallas_call(
        _vq_argmin_body,
        grid=(b, nj),
        in_specs=[
            pl.BlockSpec((1, d, HW_TILE), lambda bi, ji: (bi, 0, ji)),
            pl.BlockSpec((KCODES, DIM), lambda bi, ji: (0, 0)),
        ],
        out_specs=[
            pl.BlockSpec((1, 1, 1, HW_TILE), lambda bi, ji: (bi, ji, 0, 0)),
            pl.BlockSpec((1, 1), lambda bi, ji: (0, 0)),
        ],
        out_shape=[
            jax.ShapeDtypeStruct((b, nj, 1, HW_TILE), jnp.int32),
            jax.ShapeDtypeStruct((1, 1), jnp.float32),
        ],
    )(z3, embedding)


@functools.cache
def _make_sc_gather(n_total):
    info = plsc.get_sparse_core_info()
    nw = info.num_cores * info.num_subcores
    b_per_w = n_total // nw
    mesh = plsc.VectorSubcoreMesh(core_axis_name="c", subcore_axis_name="s")

    @functools.partial(
        pl.kernel, mesh=mesh,
        compiler_params=pltpu.CompilerParams(use_tc_tiling_on_sc=False),
        out_type=jax.ShapeDtypeStruct((n_total, DIM), jnp.float32),
        scratch_types=[
            pltpu.VMEM((b_per_w,), jnp.int32),
            pltpu.VMEM((b_per_w, DIM), jnp.float32),
            pltpu.SemaphoreType.DMA,
        ],
    )
    def gather_kernel(idx_hbm, table_hbm, out_hbm, idx_v, rows_v, sem):
        wid = lax.axis_index("s") * info.num_cores + lax.axis_index("c")
        base = wid * b_per_w
        pltpu.sync_copy(idx_hbm.at[pl.ds(base, b_per_w)], idx_v)
        pltpu.async_copy(table_hbm.at[idx_v], rows_v, sem).wait()
        pltpu.sync_copy(rows_v, out_hbm.at[pl.ds(base, b_per_w)])

    return gather_kernel


def kernel(z_e, embedding):
    b, d, h, w = z_e.shape
    n = b * h * w
    z3 = z_e.reshape(b, d, h * w)
    idx3, lsum = _vq_argmin(z3, embedding)
    indices = idx3.reshape(n)
    z_q_flat = _make_sc_gather(n)(indices, embedding)
    z_q = jnp.transpose(z_q_flat.reshape(b, h * w, d), (0, 2, 1)).reshape(b, d, h, w)
    return (z_q, lsum[0, 0], indices.reshape(b, h, w))
